# baseline (speedup 1.0000x reference)
import jax
import jax.numpy as jnp
from jax import lax
from jax.experimental import pallas as pl
from jax.experimental.pallas import tpu as pltpu
from jax.experimental.pallas import tpu_sc as plsc

N = 10000
NP = 10240
H = 256
W = 256
TS = 32
TG = 8
NT = TG * TG
SIG_CUT = 9.5
ROWSPG = NP // 128
BAND = 336
NSUB = 32
GPS = NP // NSUB
LSTN = NP + 256


def _prep_body(m_ref, ch_ref, op_ref, fdc_ref, planes_ref, bbox_ref):
    mx = m_ref[0]
    my = m_ref[1]
    x = 0.5 * (jnp.tanh(mx) + 1.0) * W
    y = 0.5 * (jnp.tanh(my) + 1.0) * H
    l1 = ch_ref[0] + 0.5
    l2 = ch_ref[1]
    l3 = ch_ref[2] + 0.5
    cov_a = l1 * l1
    cov_b = l1 * l2
    cov_c = l2 * l2 + l3 * l3
    det = jnp.maximum(cov_a * cov_c - cov_b * cov_b, 1e-12)
    inv_det = 1.0 / det
    ca = cov_c * inv_det
    cb = -cov_b * inv_det
    cc = cov_a * inv_det
    o = jax.nn.sigmoid(op_ref[0])
    colr = jax.nn.sigmoid(fdc_ref[0])
    colg = jax.nn.sigmoid(fdc_ref[1])
    colb = jax.nn.sigmoid(fdc_ref[2])

    planes_ref[0] = x
    planes_ref[1] = y
    planes_ref[2] = 0.5 * ca
    planes_ref[3] = cb
    planes_ref[4] = 0.5 * cc
    planes_ref[5] = o
    planes_ref[6] = colr
    planes_ref[7] = colg
    planes_ref[8] = colb
    zero = jnp.zeros_like(x)
    for k in range(9, 16):
        planes_ref[k] = zero

    hx = jnp.sqrt(2.0 * SIG_CUT * cov_a) + 1.0
    hy = jnp.sqrt(2.0 * SIG_CUT * cov_c) + 1.0
    gi = (lax.broadcasted_iota(jnp.int32, (ROWSPG, 128), 0) * 128
          + lax.broadcasted_iota(jnp.int32, (ROWSPG, 128), 1))
    cover = ((x + hx >= 0.0) & (x - hx <= float(W)) &
             (y + hy >= 0.0) & (y - hy <= float(H)) & (gi < N))
    tx0 = jnp.clip(jnp.floor((x - hx) / TS).astype(jnp.int32), 0, TG - 1)
    tx1 = jnp.clip(jnp.floor((x + hx) / TS).astype(jnp.int32), 0, TG - 1)
    ty0 = jnp.clip(jnp.floor((y - hy) / TS).astype(jnp.int32), 0, TG - 1)
    ty1 = jnp.clip(jnp.floor((y + hy) / TS).astype(jnp.int32), 0, TG - 1)
    tx1 = jnp.where(cover, tx1, -1)
    bbox_ref[0] = tx0
    bbox_ref[1] = tx1
    bbox_ref[2] = ty0
    bbox_ref[3] = ty1


def _prep(means_t, chol_t, opacity, features_dc):
    def to_planes(a):
        a = jnp.pad(a, ((0, NP - N), (0, 0)))
        return a.T.reshape(a.shape[1], ROWSPG, 128)

    m = to_planes(means_t)
    ch = to_planes(chol_t)
    op = to_planes(opacity)
    fdc = to_planes(features_dc)
    planes, bbox = pl.pallas_call(
        _prep_body,
        out_shape=[
            jax.ShapeDtypeStruct((16, ROWSPG, 128), jnp.float32),
            jax.ShapeDtypeStruct((4, ROWSPG, 128), jnp.int32),
        ],
    )(m, ch, op, fdc)
    return planes, bbox


def _bin_a_body(bb0_h, bb1_h, bb2_h, bb3_h, band_h,
                b0, b1, b2, b3, band_v, cnt_s):
    cid = lax.axis_index("c")
    sid = lax.axis_index("s")
    wid = sid * 2 + cid
    base = wid * GPS
    pltpu.sync_copy(bb0_h.at[pl.ds(base, GPS)], b0)
    pltpu.sync_copy(bb1_h.at[pl.ds(base, GPS)], b1)
    pltpu.sync_copy(bb2_h.at[pl.ds(base, GPS)], b2)
    pltpu.sync_copy(bb3_h.at[pl.ds(base, GPS)], b3)
    for t in range(NT):
        cnt_s[t] = 0

    def chunk_body(ci, carry):
        v0 = b0[pl.ds(ci * 16, 16)]
        v1 = b1[pl.ds(ci * 16, 16)]
        v2 = b2[pl.ds(ci * 16, 16)]
        v3 = b3[pl.ds(ci * 16, 16)]
        for j in range(16):
            tx0 = v0[j]
            tx1 = v1[j]
            ty0 = v2[j]
            ty1 = v3[j]
            gsp = jnp.broadcast_to(base + ci * 16 + j, (16,)).astype(jnp.int32)

            def ty_body(ty, c1):
                def tx_body(tx, c2):
                    t = ty * TG + tx
                    c = cnt_s[t]
                    band_v[pl.ds(t * BAND + c, 16)] = gsp
                    cnt_s[t] = c + 1
                    return c2

                return lax.fori_loop(tx0, tx1 + 1, tx_body, c1)

            lax.fori_loop(ty0, ty1 + 1, ty_body, jnp.int32(0))
        return carry

    lax.fori_loop(0, GPS // 16, chunk_body, jnp.int32(0))
    for t in range(NT):
        band_v[pl.ds(t * BAND + 320, 16)] = (
            jnp.broadcast_to(cnt_s[t], (16,)).astype(jnp.int32))
    pltpu.sync_copy(band_v, band_h.at[pl.ds(wid * NT * BAND, NT * BAND)])


CH = 128


def _bin_b_body(band_h, counts_h, ids_h, bands_v, lst, cntv, sem):
    cid = lax.axis_index("c")
    sid = lax.axis_index("s")
    wid = sid * 2 + cid

    for k in range(2):
        tt = wid * 2 + k
        copies = [
            pltpu.make_async_copy(
                band_h.at[pl.ds(s * NT * BAND + tt * BAND, BAND)],
                bands_v.at[pl.ds(s * BAND, BAND)], sem)
            for s in range(NSUB)
        ]
        for c in copies:
            c.start()
        for c in copies:
            c.wait()
        off = jnp.int32(0)
        for s in range(NSUB):
            cseg = bands_v[pl.ds(s * BAND + 320, 16)][0]

            def cp_body(ci, o, s=s):
                v = bands_v[pl.ds(s * BAND + ci * 16, 16)]
                lst[pl.ds(o + ci * 16, 16)] = v
                return o

            lax.fori_loop(0, (cseg + 15) // 16, cp_body, off)
            off = off + cseg
        cnt = off
        padv = jnp.full((16,), NP, jnp.int32)
        for q in range(16):
            lst[pl.ds(cnt + 16 * q, 16)] = padv

        cntv[...] = jnp.broadcast_to(cnt, (16,)).astype(jnp.int32)
        pltpu.sync_copy(cntv, counts_h.at[pl.ds(tt * 16, 16)])
        pltpu.sync_copy(lst, ids_h.at[pl.ds(tt * LSTN, LSTN)])


def _bin_sc(bbox):
    bb = bbox.reshape(4, NP)
    mesh = plsc.VectorSubcoreMesh(core_axis_name="c", subcore_axis_name="s")
    bin_a = pl.kernel(
        _bin_a_body,
        out_type=jax.ShapeDtypeStruct((NSUB * NT * BAND,), jnp.int32),
        mesh=mesh,
        scratch_types=[
            pltpu.VMEM((GPS,), jnp.int32),
            pltpu.VMEM((GPS,), jnp.int32),
            pltpu.VMEM((GPS,), jnp.int32),
            pltpu.VMEM((GPS,), jnp.int32),
            pltpu.VMEM((NT * BAND,), jnp.int32),
            pltpu.SMEM((NT,), jnp.int32),
        ],
    )
    band = bin_a(bb[0], bb[1], bb[2], bb[3])
    bin_b = pl.kernel(
        _bin_b_body,
        out_type=[
            jax.ShapeDtypeStruct((NT * 16,), jnp.int32),
            jax.ShapeDtypeStruct((NT * LSTN,), jnp.int32),
        ],
        mesh=mesh,
        scratch_types=[
            pltpu.VMEM((NSUB * BAND,), jnp.int32),
            pltpu.VMEM((LSTN,), jnp.int32),
            pltpu.VMEM((16,), jnp.int32),
            pltpu.SemaphoreType.DMA,
        ],
    )
    counts16, ids = bin_b(band)
    return counts16.reshape(NT, 16), ids.reshape(NT, 1, LSTN)


def _render_body(counts_ref, bg_ref, ids_ref, params_ref, out_ref):
    t = pl.program_id(0)
    cnt = counts_ref[t, 0]
    ty = t // TG
    tx = t % TG
    sub = lax.broadcasted_iota(jnp.int32, (8, 128), 0)
    lane = lax.broadcasted_iota(jnp.int32, (8, 128), 1)
    p = sub * 128 + lane
    col = p % TS
    row = p // TS
    cx = (tx * TS).astype(jnp.float32) + col.astype(jnp.float32) + 0.5
    cy = (ty * TS).astype(jnp.float32) + row.astype(jnp.float32) + 0.5

    cxs = [cx[i:i + 1, :] for i in range(8)]
    cys = [cy[i:i + 1, :] for i in range(8)]

    def group_body(g, accs):
        new_accs = list(accs)
        for h in range(32):
            base = 256 * g + 8 * h
            rows = [params_ref[pl.ds(ids_ref[0, 0, base + j], 1), :]
                    for j in range(8)]
            par = jnp.concatenate(rows, axis=0)
            X = par[:, 0:1]
            Y = par[:, 1:2]
            A = par[:, 2:3]
            B = par[:, 3:4]
            C = par[:, 4:5]
            O = par[:, 5:6]
            cols = (par[:, 6:7], par[:, 7:8], par[:, 8:9])
            for pc in range(8):
                dx = X - cxs[pc]
                dy = Y - cys[pc]
                sigma = A * (dx * dx) + C * (dy * dy) + B * (dx * dy)
                alpha = jnp.minimum(0.999, O * jnp.exp(-sigma))
                alpha = jnp.where(sigma >= 0.0, alpha, 0.0)
                for c in range(3):
                    new_accs[c * 8 + pc] = new_accs[c * 8 + pc] + alpha * cols[c]
        return tuple(new_accs)

    zero = jnp.zeros((8, 128), jnp.float32)
    accs = tuple([zero] * 24)
    accs = lax.fori_loop(0, (cnt + 255) // 256, group_body, accs)
    for c in range(3):
        planes = [jnp.sum(accs[c * 8 + pc], axis=0, keepdims=True)
                  for pc in range(8)]
        img = jnp.concatenate(planes, axis=0) + bg_ref[c]
        out_ref[0, c] = jnp.clip(img, 0.0, 1.0)


def _render(counts, background, ids3, paramsz):
    grid_spec = pltpu.PrefetchScalarGridSpec(
        num_scalar_prefetch=2,
        grid=(NT,),
        in_specs=[
            pl.BlockSpec((1, 1, LSTN), lambda t, *_: (t, 0, 0),
                         memory_space=pltpu.SMEM),
            pl.BlockSpec((NP + 8, 16), lambda t, *_: (0, 0)),
        ],
        out_specs=pl.BlockSpec((1, 3, 8, 128), lambda t, *_: (t, 0, 0, 0)),
        scratch_shapes=[],
    )
    out = pl.pallas_call(
        _render_body,
        grid_spec=grid_spec,
        out_shape=jax.ShapeDtypeStruct((NT, 3, 8, 128), jnp.float32),
    )(counts, background, ids3, paramsz)
    out = out.reshape(TG, TG, 3, 8, 4, TS)
    out = out.transpose(2, 0, 3, 4, 1, 5).reshape(3, H, W)
    return out


def kernel(xyz, cholesky, opacity, features_dc, background, frame_index):
    means_t = jnp.take(xyz, frame_index, axis=0)
    chol_t = jnp.take(cholesky, frame_index, axis=0)
    planes, bbox = _prep(means_t, chol_t, opacity, features_dc)
    counts16, ids3 = _bin_sc(bbox)
    params16 = planes.reshape(16, NP).T
    paramsz = jnp.pad(params16, ((0, 8), (0, 0)))
    out = _render(counts16, background, ids3, paramsz)
    return out[None]


# speedup vs baseline: 1.1480x; 1.1480x over previous
import jax
import jax.numpy as jnp
from jax import lax
from jax.experimental import pallas as pl
from jax.experimental.pallas import tpu as pltpu
from jax.experimental.pallas import tpu_sc as plsc

N = 10000
NP = 10240
H = 256
W = 256
TS = 32
TG = 8
NT = TG * TG
SIG_CUT = 9.5
ROWSPG = NP // 128
BAND = 336
NSUB = 32
GPS = NP // NSUB
LSTN = NP + 128


def _prep_body(m_ref, ch_ref, op_ref, fdc_ref, planes_ref, bbox_ref):
    mx = m_ref[0]
    my = m_ref[1]
    x = 0.5 * (jnp.tanh(mx) + 1.0) * W
    y = 0.5 * (jnp.tanh(my) + 1.0) * H
    l1 = ch_ref[0] + 0.5
    l2 = ch_ref[1]
    l3 = ch_ref[2] + 0.5
    cov_a = l1 * l1
    cov_b = l1 * l2
    cov_c = l2 * l2 + l3 * l3
    det = jnp.maximum(cov_a * cov_c - cov_b * cov_b, 1e-12)
    inv_det = 1.0 / det
    ca = cov_c * inv_det
    cb = -cov_b * inv_det
    cc = cov_a * inv_det
    o = jax.nn.sigmoid(op_ref[0])
    colr = jax.nn.sigmoid(fdc_ref[0])
    colg = jax.nn.sigmoid(fdc_ref[1])
    colb = jax.nn.sigmoid(fdc_ref[2])

    planes_ref[0] = x
    planes_ref[1] = y
    planes_ref[2] = 0.5 * ca
    planes_ref[3] = cb
    planes_ref[4] = 0.5 * cc
    planes_ref[5] = o
    planes_ref[6] = colr
    planes_ref[7] = colg
    planes_ref[8] = colb
    zero = jnp.zeros_like(x)
    for k in range(9, 16):
        planes_ref[k] = zero

    hx = jnp.sqrt(2.0 * SIG_CUT * cov_a) + 1.0
    hy = jnp.sqrt(2.0 * SIG_CUT * cov_c) + 1.0
    gi = (lax.broadcasted_iota(jnp.int32, (ROWSPG, 128), 0) * 128
          + lax.broadcasted_iota(jnp.int32, (ROWSPG, 128), 1))
    cover = ((x + hx >= 0.0) & (x - hx <= float(W)) &
             (y + hy >= 0.0) & (y - hy <= float(H)) & (gi < N))
    tx0 = jnp.clip(jnp.floor((x - hx) / TS).astype(jnp.int32), 0, TG - 1)
    tx1 = jnp.clip(jnp.floor((x + hx) / TS).astype(jnp.int32), 0, TG - 1)
    ty0 = jnp.clip(jnp.floor((y - hy) / TS).astype(jnp.int32), 0, TG - 1)
    ty1 = jnp.clip(jnp.floor((y + hy) / TS).astype(jnp.int32), 0, TG - 1)
    tx1 = jnp.where(cover, tx1, -1)
    bbox_ref[0] = tx0
    bbox_ref[1] = tx1
    bbox_ref[2] = ty0
    bbox_ref[3] = ty1


def _prep(means_t, chol_t, opacity, features_dc):
    def to_planes(a):
        a = jnp.pad(a, ((0, NP - N), (0, 0)))
        return a.T.reshape(a.shape[1], ROWSPG, 128)

    m = to_planes(means_t)
    ch = to_planes(chol_t)
    op = to_planes(opacity)
    fdc = to_planes(features_dc)
    planes, bbox = pl.pallas_call(
        _prep_body,
        out_shape=[
            jax.ShapeDtypeStruct((16, ROWSPG, 128), jnp.float32),
            jax.ShapeDtypeStruct((4, ROWSPG, 128), jnp.int32),
        ],
    )(m, ch, op, fdc)
    return planes, bbox


def _bin_a_body(bb0_h, bb1_h, bb2_h, bb3_h, band_h,
                b0, b1, b2, b3, band_v, cnt_s):
    cid = lax.axis_index("c")
    sid = lax.axis_index("s")
    wid = sid * 2 + cid
    base = wid * GPS
    pltpu.sync_copy(bb0_h.at[pl.ds(base, GPS)], b0)
    pltpu.sync_copy(bb1_h.at[pl.ds(base, GPS)], b1)
    pltpu.sync_copy(bb2_h.at[pl.ds(base, GPS)], b2)
    pltpu.sync_copy(bb3_h.at[pl.ds(base, GPS)], b3)
    for t in range(NT):
        cnt_s[t] = 0

    def chunk_body(ci, carry):
        v0 = b0[pl.ds(ci * 16, 16)]
        v1 = b1[pl.ds(ci * 16, 16)]
        v2 = b2[pl.ds(ci * 16, 16)]
        v3 = b3[pl.ds(ci * 16, 16)]
        for j in range(16):
            tx0 = v0[j]
            tx1 = v1[j]
            ty0 = v2[j]
            ty1 = v3[j]
            gsp = jnp.broadcast_to(base + ci * 16 + j, (16,)).astype(jnp.int32)

            def ty_body(ty, c1):
                def tx_body(tx, c2):
                    t = ty * TG + tx
                    c = cnt_s[t]
                    band_v[pl.ds(t * BAND + c, 16)] = gsp
                    cnt_s[t] = c + 1
                    return c2

                return lax.fori_loop(tx0, tx1 + 1, tx_body, c1)

            lax.fori_loop(ty0, ty1 + 1, ty_body, jnp.int32(0))
        return carry

    lax.fori_loop(0, GPS // 16, chunk_body, jnp.int32(0))
    for t in range(NT):
        band_v[pl.ds(t * BAND + 320, 16)] = (
            jnp.broadcast_to(cnt_s[t], (16,)).astype(jnp.int32))
    pltpu.sync_copy(band_v, band_h.at[pl.ds(wid * NT * BAND, NT * BAND)])


CH = 128


def _bin_b_body(band_h, counts_h, ids_h, bands_v, lst, cntv, sem):
    cid = lax.axis_index("c")
    sid = lax.axis_index("s")
    wid = sid * 2 + cid

    for k in range(2):
        tt = wid * 2 + k
        copies = [
            pltpu.make_async_copy(
                band_h.at[pl.ds(s * NT * BAND + tt * BAND, BAND)],
                bands_v.at[pl.ds(s * BAND, BAND)], sem)
            for s in range(NSUB)
        ]
        for c in copies:
            c.start()
        for c in copies:
            c.wait()
        off = jnp.int32(0)
        for s in range(NSUB):
            cseg = bands_v[pl.ds(s * BAND + 320, 16)][0]

            def cp_body(ci, o, s=s):
                v = bands_v[pl.ds(s * BAND + ci * 16, 16)]
                lst[pl.ds(o + ci * 16, 16)] = v
                return o

            lax.fori_loop(0, (cseg + 15) // 16, cp_body, off)
            off = off + cseg
        cnt = off
        padv = jnp.full((16,), NP, jnp.int32)
        for q in range(8):
            lst[pl.ds(cnt + 16 * q, 16)] = padv

        cntv[...] = jnp.broadcast_to(cnt, (16,)).astype(jnp.int32)
        pltpu.sync_copy(cntv, counts_h.at[pl.ds(tt * 16, 16)])
        pltpu.sync_copy(lst, ids_h.at[pl.ds(tt * LSTN, LSTN)])


def _bin_sc(bbox):
    bb = bbox.reshape(4, NP)
    mesh = plsc.VectorSubcoreMesh(core_axis_name="c", subcore_axis_name="s")
    bin_a = pl.kernel(
        _bin_a_body,
        out_type=jax.ShapeDtypeStruct((NSUB * NT * BAND,), jnp.int32),
        mesh=mesh,
        scratch_types=[
            pltpu.VMEM((GPS,), jnp.int32),
            pltpu.VMEM((GPS,), jnp.int32),
            pltpu.VMEM((GPS,), jnp.int32),
            pltpu.VMEM((GPS,), jnp.int32),
            pltpu.VMEM((NT * BAND,), jnp.int32),
            pltpu.SMEM((NT,), jnp.int32),
        ],
    )
    band = bin_a(bb[0], bb[1], bb[2], bb[3])
    bin_b = pl.kernel(
        _bin_b_body,
        out_type=[
            jax.ShapeDtypeStruct((NT * 16,), jnp.int32),
            jax.ShapeDtypeStruct((NT * LSTN,), jnp.int32),
        ],
        mesh=mesh,
        scratch_types=[
            pltpu.VMEM((NSUB * BAND,), jnp.int32),
            pltpu.VMEM((LSTN,), jnp.int32),
            pltpu.VMEM((16,), jnp.int32),
            pltpu.SemaphoreType.DMA,
        ],
    )
    counts16, ids = bin_b(band)
    return counts16.reshape(NT, 16), ids.reshape(NT, 1, LSTN)


def _render_body(counts_ref, bg_ref, ids_ref, params_ref, out_ref):
    t = pl.program_id(0)
    cnt = counts_ref[t, 0]
    ty = t // TG
    tx = t % TG
    sub = lax.broadcasted_iota(jnp.int32, (8, 128), 0)
    lane = lax.broadcasted_iota(jnp.int32, (8, 128), 1)
    p = sub * 128 + lane
    col = p % TS
    row = p // TS
    cx = (tx * TS).astype(jnp.float32) + col.astype(jnp.float32) + 0.5
    cy = (ty * TS).astype(jnp.float32) + row.astype(jnp.float32) + 0.5

    cxs = [cx[i:i + 1, :] for i in range(8)]
    cys = [cy[i:i + 1, :] for i in range(8)]

    def group_body(g, accs):
        new_accs = list(accs)
        for h in range(16):
            base = 128 * g + 8 * h
            rows = [params_ref[pl.ds(ids_ref[0, 0, base + j], 1), :]
                    for j in range(8)]
            par = jnp.concatenate(rows, axis=0)
            X = par[:, 0:1]
            Y = par[:, 1:2]
            A = par[:, 2:3]
            B = par[:, 3:4]
            C = par[:, 4:5]
            O = par[:, 5:6]
            cols = (par[:, 6:7], par[:, 7:8], par[:, 8:9])
            for pc in range(8):
                dx = X - cxs[pc]
                dy = Y - cys[pc]
                sigma = A * (dx * dx) + C * (dy * dy) + B * (dx * dy)
                alpha = jnp.minimum(0.999, O * jnp.exp(-sigma))
                alpha = jnp.where(sigma >= 0.0, alpha, 0.0)
                for c in range(3):
                    new_accs[c * 8 + pc] = new_accs[c * 8 + pc] + alpha * cols[c]
        return tuple(new_accs)

    zero = jnp.zeros((8, 128), jnp.float32)
    accs = tuple([zero] * 24)
    accs = lax.fori_loop(0, (cnt + 127) // 128, group_body, accs)
    for c in range(3):
        planes = [jnp.sum(accs[c * 8 + pc], axis=0, keepdims=True)
                  for pc in range(8)]
        img = jnp.concatenate(planes, axis=0) + bg_ref[c]
        out_ref[0, c] = jnp.clip(img, 0.0, 1.0)


def _render(counts, background, ids3, paramsz):
    grid_spec = pltpu.PrefetchScalarGridSpec(
        num_scalar_prefetch=2,
        grid=(NT,),
        in_specs=[
            pl.BlockSpec((1, 1, LSTN), lambda t, *_: (t, 0, 0),
                         memory_space=pltpu.SMEM),
            pl.BlockSpec((NP + 8, 16), lambda t, *_: (0, 0)),
        ],
        out_specs=pl.BlockSpec((1, 3, 8, 128), lambda t, *_: (t, 0, 0, 0)),
        scratch_shapes=[],
    )
    out = pl.pallas_call(
        _render_body,
        grid_spec=grid_spec,
        out_shape=jax.ShapeDtypeStruct((NT, 3, 8, 128), jnp.float32),
    )(counts, background, ids3, paramsz)
    out = out.reshape(TG, TG, 3, 8, 4, TS)
    out = out.transpose(2, 0, 3, 4, 1, 5).reshape(3, H, W)
    return out


def kernel(xyz, cholesky, opacity, features_dc, background, frame_index):
    means_t = jnp.take(xyz, frame_index, axis=0)
    chol_t = jnp.take(cholesky, frame_index, axis=0)
    planes, bbox = _prep(means_t, chol_t, opacity, features_dc)
    counts16, ids3 = _bin_sc(bbox)
    params16 = planes.reshape(16, NP).T
    paramsz = jnp.pad(params16, ((0, 8), (0, 0)))
    out = _render(counts16, background, ids3, paramsz)
    return out[None]
